# baseline (speedup 1.0000x reference)
import functools

import jax
import jax.numpy as jnp
from jax import lax
from jax.experimental import pallas as pl
from jax.experimental.pallas import tpu as pltpu
from jax.experimental.pallas import tpu_sc as plsc

N = 10000
NP = 10240
E = 640000
D = 128
H = 128
HH = 64
L = 4
G = 128
EPS = 1e-5

NTILE = 16
CHUNK = 128
NCHUNK = 320
IBLK = 160
ET = NCHUNK * CHUNK
EPAD = NTILE * ET - E
ROWS = NP // NTILE

BLK = 640
NBLK = NP // BLK


def _sc_deg_body(col_hbm, ones_hbm, zeros_hbm, deg_hbm, col_v, ones_v, acc):
    c = lax.axis_index("c")
    s = lax.axis_index("s")

    @pl.when(c == 0)
    def _():
        pltpu.sync_copy(col_hbm.at[s], col_v)
        pltpu.sync_copy(ones_hbm, ones_v)
        pltpu.sync_copy(zeros_hbm.at[pl.ds(s * ROWS, ROWS)],
                        acc.at[pl.ds(s * ROWS, ROWS)])
        plsc.subcore_barrier()

        def body(j, carry):
            pltpu.sync_copy(ones_v, acc.at[col_v.at[j]], add=True)
            return carry

        lax.fori_loop(0, NCHUNK, body, 0)
        plsc.subcore_barrier()
        pltpu.sync_copy(acc.at[pl.ds(s * ROWS, ROWS)],
                        deg_hbm.at[pl.ds(s * ROWS, ROWS)])


def _sc_agg_body(p0_hbm, p1_hbm, row_hbm, col_hbm, q0_hbm, q1_hbm,
                 row_v, col_v, g0, g1, g2, g3, acc,
                 sg0, sg1, sg2, sg3, ss0, ss1, ss2, ss3):
    c = lax.axis_index("c")
    s = lax.axis_index("s")
    gbufs = (g0, g1, g2, g3)
    semg = (sg0, sg1, sg2, sg3)
    sems = (ss0, ss1, ss2, ss3)

    def half(p_hbm, q_hbm):
        pltpu.sync_copy(p_hbm.at[pl.ds(s * ROWS, ROWS)],
                        acc.at[pl.ds(s * ROWS, ROWS)])
        plsc.subcore_barrier()

        for kb in range(NCHUNK // IBLK):
            pltpu.sync_copy(row_hbm.at[s].at[pl.ds(kb * IBLK, IBLK)], row_v)
            pltpu.sync_copy(col_hbm.at[s].at[pl.ds(kb * IBLK, IBLK)], col_v)
            pltpu.async_copy(p_hbm.at[row_v.at[0]], gbufs[0], semg[0])
            pltpu.async_copy(p_hbm.at[row_v.at[1]], gbufs[1], semg[1])

            def it_body(it, carry):
                for b in range(4):
                    jj = it * 4 + b
                    b2 = (b + 2) % 4
                    pltpu.make_async_copy(
                        p_hbm.at[row_v.at[jj]], gbufs[b], semg[b]).wait()
                    pltpu.async_copy(gbufs[b], acc.at[col_v.at[jj]], sems[b],
                                     add=True)

                    @pl.when(jj >= 2)
                    def _():
                        pltpu.make_async_copy(
                            gbufs[b2], acc.at[col_v.at[jj]], sems[b2]).wait()

                    @pl.when(jj + 2 < IBLK)
                    def _():
                        pltpu.async_copy(
                            p_hbm.at[row_v.at[jj + 2]], gbufs[b2], semg[b2])

                return carry

            lax.fori_loop(0, IBLK // 4, it_body, 0)
            pltpu.make_async_copy(gbufs[2], acc.at[col_v.at[0]], sems[2]).wait()
            pltpu.make_async_copy(gbufs[3], acc.at[col_v.at[0]], sems[3]).wait()

        plsc.subcore_barrier()
        pltpu.sync_copy(acc.at[pl.ds(s * ROWS, ROWS)],
                        q_hbm.at[pl.ds(s * ROWS, ROWS)])

    @pl.when(c == 0)
    def _():
        half(p0_hbm, q0_hbm)

    @pl.when(c == 1)
    def _():
        half(p1_hbm, q1_hbm)


@functools.cache
def _sc_kernels():
    mesh = plsc.VectorSubcoreMesh(core_axis_name="c", subcore_axis_name="s",
                                  num_cores=2, num_subcores=NTILE)
    params = pltpu.CompilerParams(use_tc_tiling_on_sc=False)
    sc_deg = pl.kernel(
        _sc_deg_body,
        out_type=jax.ShapeDtypeStruct((NP,), jnp.float32),
        mesh=mesh,
        compiler_params=params,
        scratch_types=[
            pltpu.VMEM((NCHUNK, CHUNK), jnp.int32),
            pltpu.VMEM((CHUNK,), jnp.float32),
            pltpu.VMEM_SHARED((NP,), jnp.float32),
        ],
    )
    sc_agg = pl.kernel(
        _sc_agg_body,
        out_type=(
            jax.ShapeDtypeStruct((NP, HH), jnp.float32),
            jax.ShapeDtypeStruct((NP, HH), jnp.float32),
        ),
        mesh=mesh,
        compiler_params=params,
        scratch_types=[
            pltpu.VMEM((IBLK, CHUNK), jnp.int32),
            pltpu.VMEM((IBLK, CHUNK), jnp.int32),
            pltpu.VMEM((CHUNK, HH), jnp.float32),
            pltpu.VMEM((CHUNK, HH), jnp.float32),
            pltpu.VMEM((CHUNK, HH), jnp.float32),
            pltpu.VMEM((CHUNK, HH), jnp.float32),
            pltpu.VMEM_SHARED((NP, HH), jnp.float32),
            pltpu.SemaphoreType.DMA,
            pltpu.SemaphoreType.DMA,
            pltpu.SemaphoreType.DMA,
            pltpu.SemaphoreType.DMA,
            pltpu.SemaphoreType.DMA,
            pltpu.SemaphoreType.DMA,
            pltpu.SemaphoreType.DMA,
            pltpu.SemaphoreType.DMA,
        ],
    )
    return sc_deg, sc_agg


def _row_mask(i):
    rowid = i * BLK + lax.broadcasted_iota(jnp.int32, (BLK, 1), 0)
    return (rowid < N).astype(jnp.float32)


def _tc_pre_body(x_ref, w_ref, deg_ref, p0_ref, p1_ref):
    dinv = lax.rsqrt(deg_ref[...] + 1.0)
    z = jnp.dot(x_ref[...], w_ref[...], preferred_element_type=jnp.float32)
    p = dinv * z
    p0_ref[...] = p[:, :HH]
    p1_ref[...] = p[:, HH:]


def _tc_pre(xp, W0, degp):
    return pl.pallas_call(
        _tc_pre_body,
        grid=(NBLK,),
        in_specs=[
            pl.BlockSpec((BLK, D), lambda i: (i, 0)),
            pl.BlockSpec((D, H), lambda i: (0, 0)),
            pl.BlockSpec((BLK, 1), lambda i: (i, 0)),
        ],
        out_specs=[
            pl.BlockSpec((BLK, HH), lambda i: (i, 0)),
            pl.BlockSpec((BLK, HH), lambda i: (i, 0)),
        ],
        out_shape=[
            jax.ShapeDtypeStruct((NP, HH), jnp.float32),
            jax.ShapeDtypeStruct((NP, HH), jnp.float32),
        ],
    )(xp, W0, degp)


def _bn_silu(y, st_ref, par_ref):
    mu = st_ref[0:1, :] * (1.0 / N)
    var = st_ref[1:2, :] * (1.0 / N) - mu * mu
    inv = lax.rsqrt(var + EPS)
    hn = par_ref[1:2, :] * (y - mu) * inv + par_ref[2:3, :]
    return hn * jax.nn.sigmoid(hn)


def _tc_mid_body(q0_ref, q1_ref, deg_ref, par_ref, w_ref,
                 p0_ref, p1_ref, st_ref):
    ph = pl.program_id(0)
    i = pl.program_id(1)
    dinv = lax.rsqrt(deg_ref[...] + 1.0)
    q = jnp.concatenate([q0_ref[...], q1_ref[...]], axis=1)
    y = dinv * q + par_ref[0:1, :]

    @pl.when(ph == 0)
    def _():
        @pl.when(i == 0)
        def _():
            st_ref[...] = jnp.zeros_like(st_ref)

        ym = y * _row_mask(i)
        st_ref[0:1, :] = st_ref[0:1, :] + jnp.sum(ym, axis=0, keepdims=True)
        st_ref[1:2, :] = st_ref[1:2, :] + jnp.sum(ym * ym, axis=0,
                                                  keepdims=True)
        p0_ref[...] = q0_ref[...]
        p1_ref[...] = q1_ref[...]

    @pl.when(ph == 1)
    def _():
        hs = _bn_silu(y, st_ref, par_ref)
        z = jnp.dot(hs, w_ref[...], preferred_element_type=jnp.float32)
        p = dinv * z
        p0_ref[...] = p[:, :HH]
        p1_ref[...] = p[:, HH:]


def _mid_out_map(ph, i):
    return (i * ph + NBLK * (1 - ph), 0)


def _tc_mid(q0, q1, degp, par, Wn):
    p0x, p1x = pl.pallas_call(
        _tc_mid_body,
        grid=(2, NBLK),
        in_specs=[
            pl.BlockSpec((BLK, HH), lambda ph, i: (i, 0)),
            pl.BlockSpec((BLK, HH), lambda ph, i: (i, 0)),
            pl.BlockSpec((BLK, 1), lambda ph, i: (i, 0)),
            pl.BlockSpec((8, H), lambda ph, i: (0, 0)),
            pl.BlockSpec((H, H), lambda ph, i: (0, 0)),
        ],
        out_specs=[
            pl.BlockSpec((BLK, HH), _mid_out_map),
            pl.BlockSpec((BLK, HH), _mid_out_map),
        ],
        out_shape=[
            jax.ShapeDtypeStruct((NP + BLK, HH), jnp.float32),
            jax.ShapeDtypeStruct((NP + BLK, HH), jnp.float32),
        ],
        scratch_shapes=[pltpu.VMEM((8, H), jnp.float32)],
    )(q0, q1, degp, par, Wn)
    return p0x[:NP], p1x[:NP]


def _tc_post_body(q0_ref, q1_ref, deg_ref, par_ref, batch_ref,
                  fc1_ref, fcp_ref, out_ref, st_ref, pool_ref, cnt_ref):
    ph = pl.program_id(0)
    i = pl.program_id(1)
    dinv = lax.rsqrt(deg_ref[...] + 1.0)
    q = jnp.concatenate([q0_ref[...], q1_ref[...]], axis=1)
    y = dinv * q + par_ref[0:1, :]

    @pl.when(ph == 0)
    def _():
        @pl.when(i == 0)
        def _():
            st_ref[...] = jnp.zeros_like(st_ref)
            out_ref[...] = jnp.zeros_like(out_ref)

        ym = y * _row_mask(i)
        st_ref[0:1, :] = st_ref[0:1, :] + jnp.sum(ym, axis=0, keepdims=True)
        st_ref[1:2, :] = st_ref[1:2, :] + jnp.sum(ym * ym, axis=0,
                                                  keepdims=True)

    @pl.when(ph == 1)
    def _():
        @pl.when(i == 0)
        def _():
            pool_ref[...] = jnp.zeros_like(pool_ref)
            cnt_ref[...] = jnp.zeros_like(cnt_ref)

        h4 = _bn_silu(y, st_ref, par_ref)
        gid = lax.broadcasted_iota(jnp.int32, (BLK, G), 1).astype(jnp.float32)
        onehot = (batch_ref[...] == gid).astype(jnp.float32)
        dn = (((0,), (0,)), ((), ()))
        pool_ref[...] = pool_ref[...] + lax.dot_general(
            onehot, h4, dn, preferred_element_type=jnp.float32)
        cnt_ref[...] = cnt_ref[...] + lax.dot_general(
            onehot, jnp.ones((BLK, H), jnp.float32), dn,
            preferred_element_type=jnp.float32)

        @pl.when(i == NBLK - 1)
        def _():
            cnt0 = cnt_ref[:, 0:1]
            pooled = jnp.where(cnt0 > 0.0,
                               pool_ref[...] / jnp.maximum(cnt0, 1.0), 0.0)
            z1 = jnp.dot(pooled, fc1_ref[...],
                         preferred_element_type=jnp.float32) + fcp_ref[0:1, :HH]
            z1 = z1 * jax.nn.sigmoid(z1)
            o = jnp.sum(z1 * fcp_ref[1:2, :HH], axis=1, keepdims=True)
            out_ref[...] = jax.nn.sigmoid(o + fcp_ref[2, 0])


def _tc_post(q0, q1, degp, par, batch_f, fc1_W, fcp):
    return pl.pallas_call(
        _tc_post_body,
        grid=(2, NBLK),
        in_specs=[
            pl.BlockSpec((BLK, HH), lambda ph, i: (i, 0)),
            pl.BlockSpec((BLK, HH), lambda ph, i: (i, 0)),
            pl.BlockSpec((BLK, 1), lambda ph, i: (i, 0)),
            pl.BlockSpec((8, H), lambda ph, i: (0, 0)),
            pl.BlockSpec((BLK, 1), lambda ph, i: (i, 0)),
            pl.BlockSpec((H, HH), lambda ph, i: (0, 0)),
            pl.BlockSpec((8, H), lambda ph, i: (0, 0)),
        ],
        out_specs=pl.BlockSpec((G, 1), lambda ph, i: (0, 0)),
        out_shape=jax.ShapeDtypeStruct((G, 1), jnp.float32),
        scratch_shapes=[
            pltpu.VMEM((8, H), jnp.float32),
            pltpu.VMEM((G, H), jnp.float32),
            pltpu.VMEM((G, H), jnp.float32),
        ],
    )(q0, q1, degp, par, batch_f, fc1_W, fcp)


def kernel(x, edge_index, batch, Ws, bs, gammas, betas,
           fc1_W, fc1_b, fc2_W, fc2_b):
    xp = jnp.pad(x, ((0, NP - N), (0, 0)))
    pad_iota = jnp.arange(EPAD, dtype=edge_index.dtype)
    row = jnp.concatenate([edge_index[0], pad_iota % N])
    col = jnp.concatenate([edge_index[1], N + pad_iota % (NP - N)])
    row_r = row.reshape(NTILE, NCHUNK, CHUNK)
    col_r = col.reshape(NTILE, NCHUNK, CHUNK)

    sc_deg, sc_agg = _sc_kernels()
    ones_h = jnp.ones((CHUNK,), jnp.float32)
    zeros_h = jnp.zeros((NP,), jnp.float32)
    deg = sc_deg(col_r, ones_h, zeros_h)
    degp = deg.reshape(NP, 1)

    pars = [
        jnp.concatenate([bs[l][None], gammas[l][None], betas[l][None],
                         jnp.zeros((5, H), jnp.float32)], axis=0)
        for l in range(L)
    ]
    fcp = (jnp.zeros((8, H), jnp.float32)
           .at[0, :HH].set(fc1_b)
           .at[1, :HH].set(fc2_W[:, 0])
           .at[2, 0].set(fc2_b[0]))
    batch_f = jnp.concatenate(
        [batch, jnp.full((NP - N,), G, batch.dtype)]
    ).astype(jnp.float32).reshape(NP, 1)

    p0, p1 = _tc_pre(xp, Ws[0], degp)
    q0 = q1 = None
    for l in range(L):
        q0, q1 = sc_agg(p0, p1, row_r, col_r)
        if l < L - 1:
            p0, p1 = _tc_mid(q0, q1, degp, pars[l], Ws[l + 1])
    return _tc_post(q0, q1, degp, pars[L - 1], batch_f, fc1_W, fcp)


# speedup vs baseline: 1.1659x; 1.1659x over previous
import functools

import jax
import jax.numpy as jnp
from jax import lax
from jax.experimental import pallas as pl
from jax.experimental.pallas import tpu as pltpu
from jax.experimental.pallas import tpu_sc as plsc

N = 10000
NP = 10240
E = 640000
D = 128
H = 128
HH = 64
L = 4
G = 128
EPS = 1e-5

NTILE = 16
CHUNK = 128
NCHUNK = 320
NRING = 5
IBLK = 80
ET = NCHUNK * CHUNK
EPAD = NTILE * ET - E
ROWS = NP // NTILE

BLK = 1280
NBLK = NP // BLK


def _sc_deg_body(col_hbm, ones_hbm, zeros_hbm, deg_hbm, col_v, ones_v, acc):
    c = lax.axis_index("c")
    s = lax.axis_index("s")

    @pl.when(c == 0)
    def _():
        pltpu.sync_copy(col_hbm.at[s], col_v)
        pltpu.sync_copy(ones_hbm, ones_v)
        pltpu.sync_copy(zeros_hbm.at[pl.ds(s * ROWS, ROWS)],
                        acc.at[pl.ds(s * ROWS, ROWS)])
        plsc.subcore_barrier()

        def body(j, carry):
            pltpu.sync_copy(ones_v, acc.at[col_v.at[j]], add=True)
            return carry

        lax.fori_loop(0, NCHUNK, body, 0)
        plsc.subcore_barrier()
        pltpu.sync_copy(acc.at[pl.ds(s * ROWS, ROWS)],
                        deg_hbm.at[pl.ds(s * ROWS, ROWS)])


def _sc_agg_body(p0_hbm, p1_hbm, row_hbm, col_hbm, q0_hbm, q1_hbm,
                 row_v, col_v, g0, g1, g2, g3, g4, acc,
                 sg0, sg1, sg2, sg3, sg4, ss0, ss1, ss2, ss3, ss4):
    c = lax.axis_index("c")
    s = lax.axis_index("s")
    gbufs = (g0, g1, g2, g3, g4)
    semg = (sg0, sg1, sg2, sg3, sg4)
    sems = (ss0, ss1, ss2, ss3, ss4)

    def half(p_hbm, q_hbm):
        pltpu.sync_copy(p_hbm.at[pl.ds(s * ROWS, ROWS)],
                        acc.at[pl.ds(s * ROWS, ROWS)])
        plsc.subcore_barrier()

        for kb in range(NCHUNK // IBLK):
            pltpu.sync_copy(row_hbm.at[s].at[pl.ds(kb * IBLK, IBLK)], row_v)
            pltpu.sync_copy(col_hbm.at[s].at[pl.ds(kb * IBLK, IBLK)], col_v)
            for b in range(3):
                pltpu.async_copy(p_hbm.at[row_v.at[b]], gbufs[b], semg[b])

            def it_body(it, carry):
                for b in range(NRING):
                    jj = it * NRING + b
                    b3 = (b + 3) % NRING
                    pltpu.make_async_copy(
                        p_hbm.at[row_v.at[jj]], gbufs[b], semg[b]).wait()
                    pltpu.async_copy(gbufs[b], acc.at[col_v.at[jj]], sems[b],
                                     add=True)

                    @pl.when(jj >= 2)
                    def _():
                        pltpu.make_async_copy(
                            gbufs[b3], acc.at[col_v.at[jj]], sems[b3]).wait()

                    @pl.when(jj + 3 < IBLK)
                    def _():
                        pltpu.async_copy(
                            p_hbm.at[row_v.at[jj + 3]], gbufs[b3], semg[b3])

                return carry

            lax.fori_loop(0, IBLK // NRING, it_body, 0)
            pltpu.make_async_copy(
                gbufs[(IBLK - 2) % NRING], acc.at[col_v.at[0]],
                sems[(IBLK - 2) % NRING]).wait()
            pltpu.make_async_copy(
                gbufs[(IBLK - 1) % NRING], acc.at[col_v.at[0]],
                sems[(IBLK - 1) % NRING]).wait()

        plsc.subcore_barrier()
        pltpu.sync_copy(acc.at[pl.ds(s * ROWS, ROWS)],
                        q_hbm.at[pl.ds(s * ROWS, ROWS)])

    @pl.when(c == 0)
    def _():
        half(p0_hbm, q0_hbm)

    @pl.when(c == 1)
    def _():
        half(p1_hbm, q1_hbm)


@functools.cache
def _sc_kernels():
    mesh = plsc.VectorSubcoreMesh(core_axis_name="c", subcore_axis_name="s",
                                  num_cores=2, num_subcores=NTILE)
    params = pltpu.CompilerParams(use_tc_tiling_on_sc=False)
    sc_deg = pl.kernel(
        _sc_deg_body,
        out_type=jax.ShapeDtypeStruct((NP,), jnp.float32),
        mesh=mesh,
        compiler_params=params,
        scratch_types=[
            pltpu.VMEM((NCHUNK, CHUNK), jnp.int32),
            pltpu.VMEM((CHUNK,), jnp.float32),
            pltpu.VMEM_SHARED((NP,), jnp.float32),
        ],
    )
    sc_agg = pl.kernel(
        _sc_agg_body,
        out_type=(
            jax.ShapeDtypeStruct((NP, HH), jnp.float32),
            jax.ShapeDtypeStruct((NP, HH), jnp.float32),
        ),
        mesh=mesh,
        compiler_params=params,
        scratch_types=(
            [pltpu.VMEM((IBLK, CHUNK), jnp.int32)] * 2
            + [pltpu.VMEM((CHUNK, HH), jnp.float32)] * NRING
            + [pltpu.VMEM_SHARED((NP, HH), jnp.float32)]
            + [pltpu.SemaphoreType.DMA] * (2 * NRING)
        ),
    )
    return sc_deg, sc_agg


def _row_mask(i):
    rowid = i * BLK + lax.broadcasted_iota(jnp.int32, (BLK, 1), 0)
    return (rowid < N).astype(jnp.float32)


def _tc_pre_body(x_ref, w_ref, deg_ref, p0_ref, p1_ref):
    dinv = lax.rsqrt(deg_ref[...] + 1.0)
    z = jnp.dot(x_ref[...], w_ref[...], preferred_element_type=jnp.float32)
    p = dinv * z
    p0_ref[...] = p[:, :HH]
    p1_ref[...] = p[:, HH:]


def _tc_pre(xp, W0, degp):
    return pl.pallas_call(
        _tc_pre_body,
        grid=(NBLK,),
        in_specs=[
            pl.BlockSpec((BLK, D), lambda i: (i, 0)),
            pl.BlockSpec((D, H), lambda i: (0, 0)),
            pl.BlockSpec((BLK, 1), lambda i: (i, 0)),
        ],
        out_specs=[
            pl.BlockSpec((BLK, HH), lambda i: (i, 0)),
            pl.BlockSpec((BLK, HH), lambda i: (i, 0)),
        ],
        out_shape=[
            jax.ShapeDtypeStruct((NP + BLK, HH), jnp.float32),
            jax.ShapeDtypeStruct((NP + BLK, HH), jnp.float32),
        ],
    )(xp, W0, degp)


def _bn_silu(y, st_ref, par_ref):
    mu = st_ref[0:1, :] * (1.0 / N)
    var = st_ref[1:2, :] * (1.0 / N) - mu * mu
    inv = lax.rsqrt(var + EPS)
    hn = par_ref[1:2, :] * (y - mu) * inv + par_ref[2:3, :]
    return hn * jax.nn.sigmoid(hn)


def _tc_mid_body(q0_ref, q1_ref, deg_ref, par_ref, w_ref,
                 p0_ref, p1_ref, st_ref):
    ph = pl.program_id(0)
    i = pl.program_id(1)
    dinv = lax.rsqrt(deg_ref[...] + 1.0)
    q = jnp.concatenate([q0_ref[...], q1_ref[...]], axis=1)
    y = dinv * q + par_ref[0:1, :]

    @pl.when(ph == 0)
    def _():
        @pl.when(i == 0)
        def _():
            st_ref[...] = jnp.zeros_like(st_ref)

        ym = y * _row_mask(i)
        st_ref[0:1, :] = st_ref[0:1, :] + jnp.sum(ym, axis=0, keepdims=True)
        st_ref[1:2, :] = st_ref[1:2, :] + jnp.sum(ym * ym, axis=0,
                                                  keepdims=True)
        p0_ref[...] = q0_ref[...]
        p1_ref[...] = q1_ref[...]

    @pl.when(ph == 1)
    def _():
        hs = _bn_silu(y, st_ref, par_ref)
        z = jnp.dot(hs, w_ref[...], preferred_element_type=jnp.float32)
        p = dinv * z
        p0_ref[...] = p[:, :HH]
        p1_ref[...] = p[:, HH:]


def _mid_out_map(ph, i):
    return (i * ph + NBLK * (1 - ph), 0)


def _tc_mid(q0, q1, degp, par, Wn):
    p0x, p1x = pl.pallas_call(
        _tc_mid_body,
        grid=(2, NBLK),
        in_specs=[
            pl.BlockSpec((BLK, HH), lambda ph, i: (i, 0)),
            pl.BlockSpec((BLK, HH), lambda ph, i: (i, 0)),
            pl.BlockSpec((BLK, 1), lambda ph, i: (i, 0)),
            pl.BlockSpec((8, H), lambda ph, i: (0, 0)),
            pl.BlockSpec((H, H), lambda ph, i: (0, 0)),
        ],
        out_specs=[
            pl.BlockSpec((BLK, HH), _mid_out_map),
            pl.BlockSpec((BLK, HH), _mid_out_map),
        ],
        out_shape=[
            jax.ShapeDtypeStruct((NP + BLK, HH), jnp.float32),
            jax.ShapeDtypeStruct((NP + BLK, HH), jnp.float32),
        ],
        scratch_shapes=[pltpu.VMEM((8, H), jnp.float32)],
    )(q0, q1, degp, par, Wn)
    return p0x, p1x


def _tc_post_body(q0_ref, q1_ref, deg_ref, par_ref, batch_ref,
                  fc1_ref, fcp_ref, out_ref, st_ref, pool_ref, cnt_ref):
    ph = pl.program_id(0)
    i = pl.program_id(1)
    dinv = lax.rsqrt(deg_ref[...] + 1.0)
    q = jnp.concatenate([q0_ref[...], q1_ref[...]], axis=1)
    y = dinv * q + par_ref[0:1, :]

    @pl.when(ph == 0)
    def _():
        @pl.when(i == 0)
        def _():
            st_ref[...] = jnp.zeros_like(st_ref)
            out_ref[...] = jnp.zeros_like(out_ref)

        ym = y * _row_mask(i)
        st_ref[0:1, :] = st_ref[0:1, :] + jnp.sum(ym, axis=0, keepdims=True)
        st_ref[1:2, :] = st_ref[1:2, :] + jnp.sum(ym * ym, axis=0,
                                                  keepdims=True)

    @pl.when(ph == 1)
    def _():
        @pl.when(i == 0)
        def _():
            pool_ref[...] = jnp.zeros_like(pool_ref)
            cnt_ref[...] = jnp.zeros_like(cnt_ref)

        h4 = _bn_silu(y, st_ref, par_ref)
        gid = lax.broadcasted_iota(jnp.int32, (BLK, G), 1).astype(jnp.float32)
        onehot = (batch_ref[...] == gid).astype(jnp.float32)
        dn = (((0,), (0,)), ((), ()))
        pool_ref[...] = pool_ref[...] + lax.dot_general(
            onehot, h4, dn, preferred_element_type=jnp.float32)
        cnt_ref[...] = cnt_ref[...] + lax.dot_general(
            onehot, jnp.ones((BLK, H), jnp.float32), dn,
            preferred_element_type=jnp.float32)

        @pl.when(i == NBLK - 1)
        def _():
            cnt0 = cnt_ref[:, 0:1]
            pooled = jnp.where(cnt0 > 0.0,
                               pool_ref[...] / jnp.maximum(cnt0, 1.0), 0.0)
            z1 = jnp.dot(pooled, fc1_ref[...],
                         preferred_element_type=jnp.float32) + fcp_ref[0:1, :HH]
            z1 = z1 * jax.nn.sigmoid(z1)
            o = jnp.sum(z1 * fcp_ref[1:2, :HH], axis=1, keepdims=True)
            out_ref[...] = jax.nn.sigmoid(o + fcp_ref[2, 0])


def _tc_post(q0, q1, degp, par, batch_f, fc1_W, fcp):
    return pl.pallas_call(
        _tc_post_body,
        grid=(2, NBLK),
        in_specs=[
            pl.BlockSpec((BLK, HH), lambda ph, i: (i, 0)),
            pl.BlockSpec((BLK, HH), lambda ph, i: (i, 0)),
            pl.BlockSpec((BLK, 1), lambda ph, i: (i, 0)),
            pl.BlockSpec((8, H), lambda ph, i: (0, 0)),
            pl.BlockSpec((BLK, 1), lambda ph, i: (i, 0)),
            pl.BlockSpec((H, HH), lambda ph, i: (0, 0)),
            pl.BlockSpec((8, H), lambda ph, i: (0, 0)),
        ],
        out_specs=pl.BlockSpec((G, 1), lambda ph, i: (0, 0)),
        out_shape=jax.ShapeDtypeStruct((G, 1), jnp.float32),
        scratch_shapes=[
            pltpu.VMEM((8, H), jnp.float32),
            pltpu.VMEM((G, H), jnp.float32),
            pltpu.VMEM((G, H), jnp.float32),
        ],
    )(q0, q1, degp, par, batch_f, fc1_W, fcp)


def kernel(x, edge_index, batch, Ws, bs, gammas, betas,
           fc1_W, fc1_b, fc2_W, fc2_b):
    xp = jnp.pad(x, ((0, NP - N), (0, 0)))
    pad_iota = jnp.arange(EPAD, dtype=edge_index.dtype)
    row = jnp.concatenate([edge_index[0], pad_iota % N])
    col = jnp.concatenate([edge_index[1], N + pad_iota % (NP - N)])
    row_r = row.reshape(NTILE, NCHUNK, CHUNK)
    col_r = col.reshape(NTILE, NCHUNK, CHUNK)

    sc_deg, sc_agg = _sc_kernels()
    ones_h = jnp.ones((CHUNK,), jnp.float32)
    zeros_h = jnp.zeros((NP,), jnp.float32)
    deg = sc_deg(col_r, ones_h, zeros_h)
    degp = deg.reshape(NP, 1)

    pars = [
        jnp.concatenate([bs[l][None], gammas[l][None], betas[l][None],
                         jnp.zeros((5, H), jnp.float32)], axis=0)
        for l in range(L)
    ]
    fcp = (jnp.zeros((8, H), jnp.float32)
           .at[0, :HH].set(fc1_b)
           .at[1, :HH].set(fc2_W[:, 0])
           .at[2, 0].set(fc2_b[0]))
    batch_f = jnp.concatenate(
        [batch, jnp.full((NP - N,), G, batch.dtype)]
    ).astype(jnp.float32).reshape(NP, 1)

    p0, p1 = _tc_pre(xp, Ws[0], degp)
    q0 = q1 = None
    for l in range(L):
        q0, q1 = sc_agg(p0, p1, row_r, col_r)
        if l < L - 1:
            p0, p1 = _tc_mid(q0, q1, degp, pars[l], Ws[l + 1])
    return _tc_post(q0, q1, degp, pars[L - 1], batch_f, fc1_W, fcp)


# speedup vs baseline: 1.3195x; 1.1317x over previous
import functools

import jax
import jax.numpy as jnp
from jax import lax
from jax.experimental import pallas as pl
from jax.experimental.pallas import tpu as pltpu
from jax.experimental.pallas import tpu_sc as plsc

N = 10000
NP = 10240
E = 640000
D = 128
H = 128
HH = 64
L = 4
G = 128
EPS = 1e-5

NTILE = 16
CHUNK = 128
NCHUNK = 320
NRING = 5
IBLK = 80
ET = NCHUNK * CHUNK
EPAD = NTILE * ET - E
ROWS = NP // NTILE

NP2 = NP // 2
BLK = 640
NBLK = NP2 // BLK


def _sc_deg_body(col_hbm, ones_hbm, zeros_hbm, deg_hbm, col_v, ones_v, acc,
                 sd0, sd1, sd2, sd3):
    c = lax.axis_index("c")
    s = lax.axis_index("s")
    sems = (sd0, sd1, sd2, sd3)

    @pl.when(c == 0)
    def _():
        pltpu.sync_copy(col_hbm.at[s], col_v)
        pltpu.sync_copy(ones_hbm, ones_v)
        pltpu.sync_copy(zeros_hbm.at[pl.ds(s * ROWS, ROWS)],
                        acc.at[pl.ds(s * ROWS, ROWS)])
        plsc.subcore_barrier()

        def body(it, carry):
            for b in range(4):
                jj = it * 4 + b

                @pl.when(jj >= 4)
                def _():
                    pltpu.make_async_copy(
                        ones_v, acc.at[col_v.at[0]], sems[b]).wait()

                pltpu.async_copy(ones_v, acc.at[col_v.at[jj]], sems[b],
                                 add=True)
            return carry

        lax.fori_loop(0, NCHUNK // 4, body, 0)
        for b in range(4):
            pltpu.make_async_copy(ones_v, acc.at[col_v.at[0]], sems[b]).wait()
        plsc.subcore_barrier()
        pltpu.sync_copy(acc.at[pl.ds(s * ROWS, ROWS)],
                        deg_hbm.at[pl.ds(s * ROWS, ROWS)])


def _sc_agg_body(p0_hbm, p1_hbm, row_hbm, col_hbm, q0_hbm, q1_hbm,
                 row_v, col_v, g0, g1, g2, g3, g4, acc,
                 sg0, sg1, sg2, sg3, sg4, ss0, ss1, ss2, ss3, ss4):
    c = lax.axis_index("c")
    s = lax.axis_index("s")
    gbufs = (g0, g1, g2, g3, g4)
    semg = (sg0, sg1, sg2, sg3, sg4)
    sems = (ss0, ss1, ss2, ss3, ss4)

    def half(p_hbm, q_hbm):
        pltpu.sync_copy(p_hbm.at[pl.ds(s * ROWS, ROWS)],
                        acc.at[pl.ds(s * ROWS, ROWS)])
        plsc.subcore_barrier()

        for kb in range(NCHUNK // IBLK):
            pltpu.sync_copy(row_hbm.at[s].at[pl.ds(kb * IBLK, IBLK)], row_v)
            pltpu.sync_copy(col_hbm.at[s].at[pl.ds(kb * IBLK, IBLK)], col_v)
            for b in range(3):
                pltpu.async_copy(p_hbm.at[row_v.at[b]], gbufs[b], semg[b])

            def it_body(it, carry):
                for b in range(NRING):
                    jj = it * NRING + b
                    b3 = (b + 3) % NRING
                    pltpu.make_async_copy(
                        p_hbm.at[row_v.at[jj]], gbufs[b], semg[b]).wait()
                    pltpu.async_copy(gbufs[b], acc.at[col_v.at[jj]], sems[b],
                                     add=True)

                    @pl.when(jj >= 2)
                    def _():
                        pltpu.make_async_copy(
                            gbufs[b3], acc.at[col_v.at[jj]], sems[b3]).wait()

                    @pl.when(jj + 3 < IBLK)
                    def _():
                        pltpu.async_copy(
                            p_hbm.at[row_v.at[jj + 3]], gbufs[b3], semg[b3])

                return carry

            lax.fori_loop(0, IBLK // NRING, it_body, 0)
            pltpu.make_async_copy(
                gbufs[(IBLK - 2) % NRING], acc.at[col_v.at[0]],
                sems[(IBLK - 2) % NRING]).wait()
            pltpu.make_async_copy(
                gbufs[(IBLK - 1) % NRING], acc.at[col_v.at[0]],
                sems[(IBLK - 1) % NRING]).wait()

        plsc.subcore_barrier()
        pltpu.sync_copy(acc.at[pl.ds(s * ROWS, ROWS)],
                        q_hbm.at[pl.ds(s * ROWS, ROWS)])

    @pl.when(c == 0)
    def _():
        half(p0_hbm, q0_hbm)

    @pl.when(c == 1)
    def _():
        half(p1_hbm, q1_hbm)


@functools.cache
def _sc_kernels():
    mesh = plsc.VectorSubcoreMesh(core_axis_name="c", subcore_axis_name="s",
                                  num_cores=2, num_subcores=NTILE)
    params = pltpu.CompilerParams(use_tc_tiling_on_sc=False)
    sc_deg = pl.kernel(
        _sc_deg_body,
        out_type=jax.ShapeDtypeStruct((NP,), jnp.float32),
        mesh=mesh,
        compiler_params=params,
        scratch_types=(
            [pltpu.VMEM((NCHUNK, CHUNK), jnp.int32),
             pltpu.VMEM((CHUNK,), jnp.float32),
             pltpu.VMEM_SHARED((NP,), jnp.float32)]
            + [pltpu.SemaphoreType.DMA] * 4
        ),
    )
    sc_agg = pl.kernel(
        _sc_agg_body,
        out_type=(
            jax.ShapeDtypeStruct((NP, HH), jnp.float32),
            jax.ShapeDtypeStruct((NP, HH), jnp.float32),
        ),
        mesh=mesh,
        compiler_params=params,
        scratch_types=(
            [pltpu.VMEM((IBLK, CHUNK), jnp.int32)] * 2
            + [pltpu.VMEM((CHUNK, HH), jnp.float32)] * NRING
            + [pltpu.VMEM_SHARED((NP, HH), jnp.float32)]
            + [pltpu.SemaphoreType.DMA] * (2 * NRING)
        ),
    )
    return sc_deg, sc_agg


def _pair_mask(i):
    rowid = i * BLK + lax.broadcasted_iota(jnp.int32, (BLK, 1), 0)
    return (rowid < N // 2).astype(jnp.float32)


def _split_pairs(q0, q1):
    he = jnp.concatenate([q0[:, :HH], q1[:, :HH]], axis=1)
    ho = jnp.concatenate([q0[:, HH:], q1[:, HH:]], axis=1)
    return he, ho


def _merge_pairs(pe, po):
    p0 = jnp.concatenate([pe[:, :HH], po[:, :HH]], axis=1)
    p1 = jnp.concatenate([pe[:, HH:], po[:, HH:]], axis=1)
    return p0, p1


def _tc_pre_body(x_ref, w_ref, deg_ref, p0_ref, p1_ref):
    dinv = lax.rsqrt(deg_ref[...] + 1.0)
    ze = jnp.dot(x_ref[:, :D], w_ref[...], preferred_element_type=jnp.float32)
    zo = jnp.dot(x_ref[:, D:], w_ref[...], preferred_element_type=jnp.float32)
    p0_ref[...], p1_ref[...] = _merge_pairs(dinv[:, 0:1] * ze,
                                            dinv[:, 1:2] * zo)


def _tc_pre(xv, W0, deg2):
    return pl.pallas_call(
        _tc_pre_body,
        grid=(NBLK,),
        in_specs=[
            pl.BlockSpec((BLK, 2 * D), lambda i: (i, 0)),
            pl.BlockSpec((D, H), lambda i: (0, 0)),
            pl.BlockSpec((BLK, 2), lambda i: (i, 0)),
        ],
        out_specs=[
            pl.BlockSpec((BLK, H), lambda i: (i, 0)),
            pl.BlockSpec((BLK, H), lambda i: (i, 0)),
        ],
        out_shape=[
            jax.ShapeDtypeStruct((NP2 + BLK, H), jnp.float32),
            jax.ShapeDtypeStruct((NP2 + BLK, H), jnp.float32),
        ],
    )(xv, W0, deg2)


def _bn_silu(y, st_ref, par_ref):
    mu = st_ref[0:1, :] * (1.0 / N)
    var = st_ref[1:2, :] * (1.0 / N) - mu * mu
    inv = lax.rsqrt(var + EPS)
    hn = par_ref[1:2, :] * (y - mu) * inv + par_ref[2:3, :]
    return hn * jax.nn.sigmoid(hn)


def _tc_mid_body(q0_ref, q1_ref, deg_ref, par_ref, w_ref,
                 p0_ref, p1_ref, st_ref):
    ph = pl.program_id(0)
    i = pl.program_id(1)
    dinv = lax.rsqrt(deg_ref[...] + 1.0)
    he, ho = _split_pairs(q0_ref[...], q1_ref[...])
    ye = dinv[:, 0:1] * he + par_ref[0:1, :]
    yo = dinv[:, 1:2] * ho + par_ref[0:1, :]

    @pl.when(ph == 0)
    def _():
        @pl.when(i == 0)
        def _():
            st_ref[...] = jnp.zeros_like(st_ref)

        m = _pair_mask(i)
        ye_m = ye * m
        yo_m = yo * m
        st_ref[0:1, :] = st_ref[0:1, :] + jnp.sum(ye_m + yo_m, axis=0,
                                                  keepdims=True)
        st_ref[1:2, :] = st_ref[1:2, :] + jnp.sum(ye_m * ye_m + yo_m * yo_m,
                                                  axis=0, keepdims=True)
        p0_ref[...] = q0_ref[...]
        p1_ref[...] = q1_ref[...]

    @pl.when(ph == 1)
    def _():
        hse = _bn_silu(ye, st_ref, par_ref)
        hso = _bn_silu(yo, st_ref, par_ref)
        ze = jnp.dot(hse, w_ref[...], preferred_element_type=jnp.float32)
        zo = jnp.dot(hso, w_ref[...], preferred_element_type=jnp.float32)
        p0_ref[...], p1_ref[...] = _merge_pairs(dinv[:, 0:1] * ze,
                                                dinv[:, 1:2] * zo)


def _mid_out_map(ph, i):
    return (i * ph + NBLK * (1 - ph), 0)


def _tc_mid(q0, q1, deg2, par, Wn):
    return pl.pallas_call(
        _tc_mid_body,
        grid=(2, NBLK),
        in_specs=[
            pl.BlockSpec((BLK, H), lambda ph, i: (i, 0)),
            pl.BlockSpec((BLK, H), lambda ph, i: (i, 0)),
            pl.BlockSpec((BLK, 2), lambda ph, i: (i, 0)),
            pl.BlockSpec((8, H), lambda ph, i: (0, 0)),
            pl.BlockSpec((H, H), lambda ph, i: (0, 0)),
        ],
        out_specs=[
            pl.BlockSpec((BLK, H), _mid_out_map),
            pl.BlockSpec((BLK, H), _mid_out_map),
        ],
        out_shape=[
            jax.ShapeDtypeStruct((NP2 + BLK, H), jnp.float32),
            jax.ShapeDtypeStruct((NP2 + BLK, H), jnp.float32),
        ],
        scratch_shapes=[pltpu.VMEM((8, H), jnp.float32)],
    )(q0, q1, deg2, par, Wn)


def _tc_post_body(q0_ref, q1_ref, deg_ref, par_ref, batch_ref,
                  fc1_ref, fcp_ref, out_ref, st_ref, pool_ref, cnt_ref):
    ph = pl.program_id(0)
    i = pl.program_id(1)
    dinv = lax.rsqrt(deg_ref[...] + 1.0)
    he, ho = _split_pairs(q0_ref[...], q1_ref[...])
    ye = dinv[:, 0:1] * he + par_ref[0:1, :]
    yo = dinv[:, 1:2] * ho + par_ref[0:1, :]

    @pl.when(ph == 0)
    def _():
        @pl.when(i == 0)
        def _():
            st_ref[...] = jnp.zeros_like(st_ref)
            out_ref[...] = jnp.zeros_like(out_ref)

        m = _pair_mask(i)
        ye_m = ye * m
        yo_m = yo * m
        st_ref[0:1, :] = st_ref[0:1, :] + jnp.sum(ye_m + yo_m, axis=0,
                                                  keepdims=True)
        st_ref[1:2, :] = st_ref[1:2, :] + jnp.sum(ye_m * ye_m + yo_m * yo_m,
                                                  axis=0, keepdims=True)

    @pl.when(ph == 1)
    def _():
        @pl.when(i == 0)
        def _():
            pool_ref[...] = jnp.zeros_like(pool_ref)
            cnt_ref[...] = jnp.zeros_like(cnt_ref)

        h4e = _bn_silu(ye, st_ref, par_ref)
        h4o = _bn_silu(yo, st_ref, par_ref)
        gid = lax.broadcasted_iota(jnp.int32, (BLK, G), 1).astype(jnp.float32)
        oe = (batch_ref[:, 0:1] == gid).astype(jnp.float32)
        oo = (batch_ref[:, 1:2] == gid).astype(jnp.float32)
        dn = (((0,), (0,)), ((), ()))
        pool_ref[...] = (pool_ref[...]
                         + lax.dot_general(oe, h4e, dn,
                                           preferred_element_type=jnp.float32)
                         + lax.dot_general(oo, h4o, dn,
                                           preferred_element_type=jnp.float32))
        cnt_ref[...] = cnt_ref[...] + lax.dot_general(
            oe + oo, jnp.ones((BLK, H), jnp.float32), dn,
            preferred_element_type=jnp.float32)

        @pl.when(i == NBLK - 1)
        def _():
            cnt0 = cnt_ref[:, 0:1]
            pooled = jnp.where(cnt0 > 0.0,
                               pool_ref[...] / jnp.maximum(cnt0, 1.0), 0.0)
            z1 = jnp.dot(pooled, fc1_ref[...],
                         preferred_element_type=jnp.float32) + fcp_ref[0:1, :HH]
            z1 = z1 * jax.nn.sigmoid(z1)
            o = jnp.sum(z1 * fcp_ref[1:2, :HH], axis=1, keepdims=True)
            out_ref[...] = jax.nn.sigmoid(o + fcp_ref[2, 0])


def _tc_post(q0, q1, deg2, par, batch2, fc1_W, fcp):
    return pl.pallas_call(
        _tc_post_body,
        grid=(2, NBLK),
        in_specs=[
            pl.BlockSpec((BLK, H), lambda ph, i: (i, 0)),
            pl.BlockSpec((BLK, H), lambda ph, i: (i, 0)),
            pl.BlockSpec((BLK, 2), lambda ph, i: (i, 0)),
            pl.BlockSpec((8, H), lambda ph, i: (0, 0)),
            pl.BlockSpec((BLK, 2), lambda ph, i: (i, 0)),
            pl.BlockSpec((H, HH), lambda ph, i: (0, 0)),
            pl.BlockSpec((8, H), lambda ph, i: (0, 0)),
        ],
        out_specs=pl.BlockSpec((G, 1), lambda ph, i: (0, 0)),
        out_shape=jax.ShapeDtypeStruct((G, 1), jnp.float32),
        scratch_shapes=[
            pltpu.VMEM((8, H), jnp.float32),
            pltpu.VMEM((G, H), jnp.float32),
            pltpu.VMEM((G, H), jnp.float32),
        ],
    )(q0, q1, deg2, par, batch2, fc1_W, fcp)


def kernel(x, edge_index, batch, Ws, bs, gammas, betas,
           fc1_W, fc1_b, fc2_W, fc2_b):
    xp = jnp.pad(x, ((0, NP - N), (0, 0)))
    pad_iota = jnp.arange(EPAD, dtype=edge_index.dtype)
    row = jnp.concatenate([edge_index[0], pad_iota % N])
    col = jnp.concatenate([edge_index[1], N + pad_iota % (NP - N)])
    row_r = row.reshape(NTILE, NCHUNK, CHUNK)
    col_r = col.reshape(NTILE, NCHUNK, CHUNK)

    sc_deg, sc_agg = _sc_kernels()
    ones_h = jnp.ones((CHUNK,), jnp.float32)
    zeros_h = jnp.zeros((NP,), jnp.float32)
    deg = sc_deg(col_r, ones_h, zeros_h)
    deg2 = deg.reshape(NP2, 2)

    pars = [
        jnp.concatenate([bs[l][None], gammas[l][None], betas[l][None],
                         jnp.zeros((5, H), jnp.float32)], axis=0)
        for l in range(L)
    ]
    fcp = (jnp.zeros((8, H), jnp.float32)
           .at[0, :HH].set(fc1_b)
           .at[1, :HH].set(fc2_W[:, 0])
           .at[2, 0].set(fc2_b[0]))
    batch2 = jnp.concatenate(
        [batch, jnp.full((NP - N,), G, batch.dtype)]
    ).astype(jnp.float32).reshape(NP2, 2)

    xv = xp.reshape(NP2, 2 * D)
    P0, P1 = _tc_pre(xv, Ws[0], deg2)
    q0 = q1 = None
    for l in range(L):
        q0, q1 = sc_agg(P0.reshape(-1, HH), P1.reshape(-1, HH), row_r, col_r)
        Q0 = q0.reshape(NP2, H)
        Q1 = q1.reshape(NP2, H)
        if l < L - 1:
            P0, P1 = _tc_mid(Q0, Q1, deg2, pars[l], Ws[l + 1])
    return _tc_post(Q0, Q1, deg2, pars[L - 1], batch2, fc1_W, fcp)
